# Initial kernel scaffold; baseline (speedup 1.0000x reference)
import jax
import jax.numpy as jnp
from jax.experimental import pallas as pl


def kernel(x, table):
    raise NotImplementedError("write your pallas kernel here")


# speedup vs baseline: 8.1855x; 8.1855x over previous
import functools

import jax
import jax.numpy as jnp
from jax import lax
from jax.experimental import pallas as pl
from jax.experimental.pallas import tpu as pltpu
from jax.experimental.pallas import tpu_sc as plsc

D = 128
B_TOTAL = 4096 * 200
NC, NS = 2, 16
NW = NC * NS
BPW = B_TOTAL // NW
SUB = 128
C = 512
NSUB = C // SUB
NCH = BPW // C
ROWS_X = B_TOTAL // SUB

_mesh = plsc.VectorSubcoreMesh(core_axis_name="c", subcore_axis_name="s")


@functools.partial(
    pl.kernel,
    mesh=_mesh,
    out_type=jax.ShapeDtypeStruct((B_TOTAL, D), jnp.float32),
    scratch_types=[
        pltpu.VMEM((NSUB, SUB), jnp.int32),
        pltpu.VMEM((C, D), jnp.float32),
        pltpu.SemaphoreType.DMA,
    ],
)
def _emb_lookup(x_hbm, tab_hbm, out_hbm, idx_v, rows_v, gsem):
    wid = lax.axis_index("s") * NC + lax.axis_index("c")
    rbase = wid * (BPW // SUB)

    def chunk(g, carry):
        row = rbase + g * NSUB
        pltpu.sync_copy(x_hbm.at[pl.ds(row, NSUB)], idx_v)
        copies = [
            pltpu.async_copy(
                tab_hbm.at[idx_v.at[j]], rows_v.at[pl.ds(j * SUB, SUB)], gsem
            )
            for j in range(NSUB)
        ]
        for cp in copies:
            cp.wait()
        pltpu.sync_copy(rows_v, out_hbm.at[pl.ds(row * SUB, C)])
        return carry

    lax.fori_loop(0, NCH, chunk, 0)


def kernel(x, table):
    xf = x.astype(jnp.int32).reshape(ROWS_X, SUB)
    out = _emb_lookup(xf, table)
    return out.reshape(x.shape[0], x.shape[1], D)


# speedup vs baseline: 9.1563x; 1.1186x over previous
import functools

import jax
import jax.numpy as jnp
from jax import lax
from jax.experimental import pallas as pl
from jax.experimental.pallas import tpu as pltpu
from jax.experimental.pallas import tpu_sc as plsc

D = 128
B_TOTAL = 4096 * 200
NC, NS = 2, 16
NW = NC * NS
BPW = B_TOTAL // NW
SUB = 128
C = 256
NSUB = C // SUB
NCH = BPW // C
NPAIR = NCH // 2 - 1
ROWS_X = B_TOTAL // SUB

_mesh = plsc.VectorSubcoreMesh(core_axis_name="c", subcore_axis_name="s")


@functools.partial(
    pl.kernel,
    mesh=_mesh,
    out_type=jax.ShapeDtypeStruct((B_TOTAL, D), jnp.float32),
    scratch_types=[
        pltpu.VMEM((NSUB, SUB), jnp.int32),
        pltpu.VMEM((NSUB, SUB), jnp.int32),
        pltpu.VMEM((C, D), jnp.float32),
        pltpu.VMEM((C, D), jnp.float32),
        pltpu.SemaphoreType.DMA,
        pltpu.SemaphoreType.DMA,
        pltpu.SemaphoreType.DMA,
        pltpu.SemaphoreType.DMA,
    ],
)
def _emb_lookup(x_hbm, tab_hbm, out_hbm,
                idx0, idx1, rows0, rows1, gsem0, gsem1, osem0, osem1):
    wid = lax.axis_index("s") * NC + lax.axis_index("c")
    rx = wid * (BPW // SUB)

    def load_idx(idxv, g):
        pltpu.sync_copy(x_hbm.at[pl.ds(rx + g * NSUB, NSUB)], idxv)

    def fire_gather(idxv, rowsv, sem):
        for j in range(NSUB):
            pltpu.async_copy(
                tab_hbm.at[idxv.at[j]], rowsv.at[pl.ds(j * SUB, SUB)], sem
            )

    def drain_gather(idxv, rowsv, sem):
        for j in range(NSUB):
            pltpu.make_async_copy(
                tab_hbm.at[idxv.at[j]], rowsv.at[pl.ds(j * SUB, SUB)], sem
            ).wait()

    def fire_store(rowsv, g, sem):
        pltpu.async_copy(rowsv, out_hbm.at[pl.ds((rx + g * NSUB) * SUB, C)], sem)

    def drain_store(rowsv, g, sem):
        pltpu.make_async_copy(
            rowsv, out_hbm.at[pl.ds((rx + g * NSUB) * SUB, C)], sem
        ).wait()

    load_idx(idx0, 0)
    fire_gather(idx0, rows0, gsem0)
    load_idx(idx1, 1)
    fire_gather(idx1, rows1, gsem1)

    def body(p, carry):
        g0 = 2 * p
        g1 = g0 + 1
        drain_gather(idx0, rows0, gsem0)
        fire_store(rows0, g0, osem0)
        drain_gather(idx1, rows1, gsem1)
        fire_store(rows1, g1, osem1)
        load_idx(idx0, g0 + 2)
        drain_store(rows0, g0, osem0)
        fire_gather(idx0, rows0, gsem0)
        load_idx(idx1, g1 + 2)
        drain_store(rows1, g1, osem1)
        fire_gather(idx1, rows1, gsem1)
        return carry

    lax.fori_loop(0, NPAIR, body, 0)

    g0 = 2 * NPAIR
    g1 = g0 + 1
    drain_gather(idx0, rows0, gsem0)
    fire_store(rows0, g0, osem0)
    drain_gather(idx1, rows1, gsem1)
    fire_store(rows1, g1, osem1)
    drain_store(rows0, g0, osem0)
    drain_store(rows1, g1, osem1)


def kernel(x, table):
    xf = x.astype(jnp.int32).reshape(ROWS_X, SUB)
    out = _emb_lookup(xf, table)
    return out.reshape(x.shape[0], x.shape[1], D)
